# baseline (speedup 1.0000x reference)
import functools

import jax
import jax.numpy as jnp
from jax import lax
from jax.experimental import pallas as pl
from jax.experimental.pallas import tpu as pltpu
from jax.experimental.pallas import tpu_sc as plsc

_L = 16
_NS = 16
_S = 10
_C = 112
_U = 2 * _C
_BP = 100352
_BPW0 = 6048
_BPW1 = 224
assert _NS * (_BPW0 + _BPW1) == _BP and _BPW0 % _U == 0 and _BPW1 % _U == 0
_BPWMAX = max(_BPW0, _BPW1)


@functools.lru_cache(maxsize=None)
def _make_sc_agg(D: int):
    mesh = plsc.VectorSubcoreMesh(core_axis_name="c", subcore_axis_name="s")

    @functools.partial(
        pl.kernel,
        mesh=mesh,
        out_type=jax.ShapeDtypeStruct((_BP, D), jnp.float32),
        scratch_types=[
            pltpu.VMEM((_S * _BPWMAX,), jnp.int32),
            pltpu.VMEM((2, _C, D), jnp.float32),
            pltpu.SemaphoreType.DMA,
            pltpu.SemaphoreType.DMA,
        ],
    )
    def k(features_hbm, idx_hbm, out_hbm, idx_v, acc_v, sem0, sem1):
        cid = lax.axis_index("c")
        sid = lax.axis_index("s")
        is0 = cid == 0
        bpw = jnp.where(is0, _BPW0, _BPW1)
        w_node_base = jnp.where(
            is0, sid * _BPW0, _NS * _BPW0 + sid * _BPW1
        )
        n_pairs = bpw // _U
        sems = (sem0, sem1)

        pltpu.sync_copy(
            idx_hbm.at[pl.ds(w_node_base * _S, _S * _BPWMAX)], idx_v
        )

        def zero(buf):
            def zbody(n, c2):
                for d in range(D // _L):
                    acc_v[buf, n, pl.ds(d * _L, _L)] = jnp.zeros((_L,), jnp.float32)
                return c2

            lax.fori_loop(0, _C, zbody, 0)

        def fire(ci, buf):
            for s in range(_S):
                pltpu.async_copy(
                    features_hbm.at[idx_v.at[pl.ds(s * bpw + ci * _C, _C)]],
                    acc_v.at[buf],
                    sems[buf],
                    add=True,
                )

        def drain(buf):
            for _ in range(_S):
                pltpu.make_async_copy(
                    features_hbm.at[pl.ds(0, _C)], acc_v.at[buf], sems[buf]
                ).wait()

        def scale_store(ci, buf):
            def sbody(n, c2):
                for d in range(D // _L):
                    acc_v[buf, n, pl.ds(d * _L, _L)] = acc_v[
                        buf, n, pl.ds(d * _L, _L)
                    ] * (1.0 / _S)
                return c2

            lax.fori_loop(0, _C, sbody, 0)
            nbase = w_node_base + ci * _C
            pltpu.sync_copy(acc_v.at[buf], out_hbm.at[pl.ds(nbase, _C)])

        zero(0)
        fire(0, 0)

        def pair_body(g, carry):
            c0 = 2 * g
            zero(1)
            fire(c0 + 1, 1)
            drain(0)
            scale_store(c0, 0)

            @pl.when(g < n_pairs - 1)
            def _():
                zero(0)
                fire(c0 + 2, 0)

            drain(1)
            scale_store(c0 + 1, 1)
            return carry

        lax.fori_loop(0, n_pairs, pair_body, 0)

    return k


def kernel(nodes, neigh_idx, features):
    B, S = neigh_idx.shape
    D = features.shape[1]
    assert S == _S and B <= _BP
    idx = neigh_idx.astype(jnp.int32)
    if _BP != B:
        idx = jnp.pad(idx, ((0, _BP - B), (0, 0)))
    split = _NS * _BPW0
    part0 = idx[:split].reshape(_NS, _BPW0, S).transpose(0, 2, 1).reshape(-1)
    part1 = idx[split:].reshape(_NS, _BPW1, S).transpose(0, 2, 1).reshape(-1)
    idxT = jnp.concatenate([part0, part1])
    out = _make_sc_agg(D)(features, idxT)
    return out[:B]


# speedup vs baseline: 1.8427x; 1.8427x over previous
import functools

import jax
import jax.numpy as jnp
from jax import lax
from jax.experimental import pallas as pl
from jax.experimental.pallas import tpu as pltpu
from jax.experimental.pallas import tpu_sc as plsc

_L = 16
_NW = 32
_S = 10
_C = 80
_U = 2 * _C


@functools.lru_cache(maxsize=None)
def _make_sc_agg(B: int, D: int):
    assert B % _U == 0
    units = B // _U
    pa = -(-units // _NW)
    pb = pa - 1
    na = units - _NW * pb
    assert 0 < na <= _NW
    win = _S * _U * pa
    mesh = plsc.VectorSubcoreMesh(core_axis_name="c", subcore_axis_name="s")

    @functools.partial(
        pl.kernel,
        mesh=mesh,
        out_type=jax.ShapeDtypeStruct((B, D), jnp.float32),
        scratch_types=[
            pltpu.VMEM((win,), jnp.int32),
            pltpu.VMEM((2, _C, D), jnp.float32),
            pltpu.SemaphoreType.DMA,
            pltpu.SemaphoreType.DMA,
        ],
    )
    def k(features_hbm, idx_hbm, out_hbm, idx_v, acc_v, sem0, sem1):
        cid = lax.axis_index("c")
        sid = lax.axis_index("s")
        wid = sid * 2 + cid
        isa = wid < na
        n_pairs = jnp.where(isa, pa, pb)
        pair_base = jnp.where(isa, wid * pa, na * pa + (wid - na) * pb)
        w_node_base = pair_base * _U
        bpw = n_pairs * _U
        sems = (sem0, sem1)

        pltpu.sync_copy(idx_hbm.at[pl.ds(w_node_base * _S, win)], idx_v)

        def zero(buf):
            def zbody(n, c2):
                for d in range(D // _L):
                    acc_v[buf, n, pl.ds(d * _L, _L)] = jnp.zeros((_L,), jnp.float32)
                return c2

            lax.fori_loop(0, _C, zbody, 0)

        def fire(ci, buf):
            for s in range(_S):
                pltpu.async_copy(
                    features_hbm.at[idx_v.at[pl.ds(s * bpw + ci * _C, _C)]],
                    acc_v.at[buf],
                    sems[buf],
                    add=True,
                )

        def drain(buf):
            for _ in range(_S):
                pltpu.make_async_copy(
                    features_hbm.at[pl.ds(0, _C)], acc_v.at[buf], sems[buf]
                ).wait()

        def scale_store(ci, buf):
            def sbody(n, c2):
                for d in range(D // _L):
                    acc_v[buf, n, pl.ds(d * _L, _L)] = acc_v[
                        buf, n, pl.ds(d * _L, _L)
                    ] * (1.0 / _S)
                return c2

            lax.fori_loop(0, _C, sbody, 0)
            nbase = w_node_base + ci * _C
            pltpu.sync_copy(acc_v.at[buf], out_hbm.at[pl.ds(nbase, _C)])

        zero(0)
        fire(0, 0)

        def pair_body(g, carry):
            c0 = 2 * g
            zero(1)
            fire(c0 + 1, 1)
            drain(0)
            scale_store(c0, 0)

            @pl.when(g < n_pairs - 1)
            def _():
                zero(0)
                fire(c0 + 2, 0)

            drain(1)
            scale_store(c0 + 1, 1)
            return carry

        lax.fori_loop(0, n_pairs, pair_body, 0)

    return k, na, pa, pb


def kernel(nodes, neigh_idx, features):
    B, S = neigh_idx.shape
    D = features.shape[1]
    assert S == _S
    k, na, pa, pb = _make_sc_agg(B, D)
    nb = _NW - na
    idx = neigh_idx.astype(jnp.int32)
    split = na * pa * _U
    part_a = idx[:split].reshape(na, pa * _U, S).transpose(0, 2, 1).reshape(-1)
    part_b = idx[split:].reshape(nb, pb * _U, S).transpose(0, 2, 1).reshape(-1)
    idx_t = jnp.concatenate([part_a, part_b])
    idx_t = jnp.pad(idx_t, (0, _S * _U * (pa - pb)))
    return k(features, idx_t)
